# Initial kernel scaffold; baseline (speedup 1.0000x reference)
import jax
import jax.numpy as jnp
from jax.experimental import pallas as pl


def kernel(x, pos_embed_weight):
    raise NotImplementedError("write your pallas kernel here")


# speedup vs baseline: 1.6995x; 1.6995x over previous
import jax
import jax.numpy as jnp
from jax.experimental import pallas as pl

SEQ_BLOCK = 512


def _add_kernel(x_ref, pos_ref, o_ref):
    pos = pos_ref[...]
    o_ref[...] = x_ref[...] + pos[:, None, :]


def kernel(x, pos_embed_weight):
    S, B, D = x.shape
    pos = pos_embed_weight[:S]
    grid = (S // SEQ_BLOCK,)
    return pl.pallas_call(
        _add_kernel,
        grid=grid,
        in_specs=[
            pl.BlockSpec((SEQ_BLOCK, B, D), lambda i: (i, 0, 0)),
            pl.BlockSpec((SEQ_BLOCK, D), lambda i: (i, 0)),
        ],
        out_specs=pl.BlockSpec((SEQ_BLOCK, B, D), lambda i: (i, 0, 0)),
        out_shape=jax.ShapeDtypeStruct((S, B, D), x.dtype),
    )(x, pos)
